# baseline (speedup 1.0000x reference)
import functools

import jax
import jax.numpy as jnp
from jax import lax
from jax.experimental import pallas as pl
from jax.experimental.pallas import tpu as pltpu
from jax.experimental.pallas import tpu_sc as plsc

_L = 16


def kernel(k_cache, k, step):
    B, H, S, D = k_cache.shape
    Q = k.shape[-2]
    BH = B * H
    start = jnp.clip(jnp.asarray(step, jnp.int32) - Q, 0, S - Q)

    kc = k_cache.reshape(BH * S, D)
    kk = k.reshape(BH * Q, D)
    start_v = jnp.full((_L,), start, jnp.int32)

    info = plsc.get_sparse_core_info()
    nc, ns = info.num_cores, info.num_subcores
    nw = nc * ns
    rows_pw = (BH * Q) // nw
    bh_pw = BH // nw

    mesh = plsc.VectorSubcoreMesh(core_axis_name="c", subcore_axis_name="s")

    @functools.partial(
        pl.kernel,
        mesh=mesh,
        scratch_types=[
            pltpu.VMEM((_L,), jnp.int32),
            pltpu.VMEM((rows_pw,), jnp.int32),
            pltpu.VMEM((rows_pw, D), jnp.float32),
            pltpu.VMEM((rows_pw, D), jnp.float32),
            pltpu.SemaphoreType.DMA,
            pltpu.SemaphoreType.DMA,
            pltpu.SemaphoreType.DMA,
            pltpu.SemaphoreType.DMA,
        ],
    )
    def sc_update(cache_ref, k_ref, s_ref, st_v, idx_v, k_v, slab_v,
                  sem_s, sem_k, sem_g, sem_w):
        wid = lax.axis_index("s") * nc + lax.axis_index("c")
        base_bh = wid * bh_pw
        base_krow = wid * rows_pw
        sin = pltpu.async_copy(s_ref, st_v, sem_s)
        kin = pltpu.async_copy(
            k_ref.at[pl.ds(base_krow, rows_pw)], k_v, sem_k)
        sin.wait()
        stv = st_v[...]
        for b in range(bh_pw):
            for c in range(Q // _L):
                base = (base_bh + b) * S + c * _L
                vec = lax.iota(jnp.int32, _L) + jnp.full((_L,), base, jnp.int32)
                idx_v[pl.ds(b * Q + c * _L, _L)] = vec + stv
        gin = pltpu.async_copy(cache_ref.at[idx_v], slab_v, sem_g)
        kin.wait()
        gin.wait()

        def add_row(i, carry):
            for c in range(D // _L):
                x = k_v[i, pl.ds(c * _L, _L)]
                plsc.addupdate(slab_v.at[i, pl.ds(c * _L, _L)], x)
            return carry

        lax.fori_loop(0, rows_pw, add_row, 0, unroll=4)
        pltpu.async_copy(slab_v, cache_ref.at[idx_v], sem_w).wait()

    cache_box = jax.new_ref(kc)
    sc_update(cache_box, kk, start_v)
    return cache_box[...].reshape(B, H, S, D)


# speedup vs baseline: 1.1257x; 1.1257x over previous
import jax
import jax.numpy as jnp
from jax.experimental import pallas as pl
from jax.experimental.pallas import tpu as pltpu

_BH_B = 32
_S_B = 512


def _make_body(Q, D):
    def body(s_ref, cache_ref, k_ref, out_ref):
        b = pl.program_id(1)
        start = s_ref[0]
        b0 = start // _S_B
        o0 = start - b0 * _S_B
        hit = jnp.logical_or(b == b0, b == b0 + 1)

        @pl.when(hit)
        def _():
            kb = k_ref[...]
            kpad = jnp.concatenate(
                [kb, jnp.zeros((_BH_B, _S_B - Q, D), kb.dtype)], axis=1)
            rolled = pltpu.roll(kpad, o0, axis=1)
            r = jax.lax.broadcasted_iota(jnp.int32, kpad.shape, 1)
            mask = jnp.logical_xor(r >= o0, b != b0)
            out_ref[...] = cache_ref[...] + jnp.where(mask, rolled, 0.0)

        @pl.when(jnp.logical_not(hit))
        def _():
            out_ref[...] = cache_ref[...]

    return body


def kernel(k_cache, k, step):
    B, H, S, D = k_cache.shape
    Q = k.shape[-2]
    BH = B * H
    start = jnp.clip(jnp.asarray(step, jnp.int32) - Q, 0, S - Q)

    kc = k_cache.reshape(BH, S, D)
    kk = k.reshape(BH, Q, D)

    cache_spec = pl.BlockSpec(
        (_BH_B, _S_B, D), lambda i, j, s_ref: (i, j, 0))
    k_spec = pl.BlockSpec((_BH_B, Q, D), lambda i, j, s_ref: (i, 0, 0))

    grid_spec = pltpu.PrefetchScalarGridSpec(
        num_scalar_prefetch=1,
        grid=(BH // _BH_B, S // _S_B),
        in_specs=[cache_spec, k_spec],
        out_specs=cache_spec,
    )
    out = pl.pallas_call(
        _make_body(Q, D),
        grid_spec=grid_spec,
        out_shape=jax.ShapeDtypeStruct(kc.shape, kc.dtype),
        compiler_params=pltpu.CompilerParams(
            dimension_semantics=("parallel", "parallel"),
        ),
    )(start.reshape(1), kc, kk)
    return out.reshape(B, H, S, D)
